# Initial kernel scaffold; baseline (speedup 1.0000x reference)
import jax
import jax.numpy as jnp
from jax.experimental import pallas as pl


def kernel(idx, table, W, b):
    raise NotImplementedError("write your pallas kernel here")


# speedup vs baseline: 9.5936x; 9.5936x over previous
import functools

import jax
import jax.numpy as jnp
from jax import lax
from jax.experimental import pallas as pl
from jax.experimental.pallas import tpu as pltpu
from jax.experimental.pallas import tpu_sc as plsc

_V = 1_000_000
_D = 32
_FLAT = 16384 * 50

_ROWBLK = 8000


def _transform_body(t_ref, wt_ref, b_ref, o_ref):
    o_ref[...] = (
        jnp.dot(t_ref[...], wt_ref[...], preferred_element_type=jnp.float32)
        + b_ref[...]
    )


def _transform(table, Wt, b2d):
    n = table.shape[0]
    return pl.pallas_call(
        _transform_body,
        grid=(n // _ROWBLK,),
        in_specs=[
            pl.BlockSpec((_ROWBLK, _D), lambda i: (i, 0)),
            pl.BlockSpec((_D, _D), lambda i: (0, 0)),
            pl.BlockSpec((1, _D), lambda i: (0, 0)),
        ],
        out_specs=pl.BlockSpec((_ROWBLK, _D), lambda i: (i, 0)),
        out_shape=jax.ShapeDtypeStruct((n, _D), jnp.float32),
    )(table, Wt, b2d)


_NW = 32
_PERW = _FLAT // _NW
_CH = 512
_NCH = _PERW // _CH

@functools.cache
def _build_gather():
    mesh = plsc.VectorSubcoreMesh(core_axis_name="c", subcore_axis_name="s")

    @functools.partial(
        pl.kernel,
        mesh=mesh,
        compiler_params=pltpu.CompilerParams(use_tc_tiling_on_sc=False),
        out_type=jax.ShapeDtypeStruct((_FLAT, _D), jnp.float32),
        scratch_types=[
            pltpu.VMEM((_CH,), jnp.int32),
            pltpu.VMEM((_CH, _D), jnp.float32),
            pltpu.SemaphoreType.DMA,
        ],
    )
    def _gather(tt_hbm, idx_hbm, out_hbm, idx_v, rows_v, sem):
        wid = lax.axis_index("s") * 2 + lax.axis_index("c")
        base = wid * _PERW

        def body(i, carry):
            off = base + i * _CH
            pltpu.sync_copy(idx_hbm.at[pl.ds(off, _CH)], idx_v)
            pltpu.async_copy(tt_hbm.at[idx_v], rows_v, sem).wait()
            pltpu.sync_copy(rows_v, out_hbm.at[pl.ds(off, _CH)])
            return carry

        lax.fori_loop(0, _NCH, body, 0)

    return _gather


def kernel(idx, table, W, b):
    ttable = _transform(table, W.T, b.reshape(1, _D))
    flat = _build_gather()(ttable, idx.reshape(-1))
    return flat.reshape(idx.shape[0], idx.shape[1], _D)


# speedup vs baseline: 9.8743x; 1.0293x over previous
import functools

import jax
import jax.numpy as jnp
from jax import lax
from jax.experimental import pallas as pl
from jax.experimental.pallas import tpu as pltpu
from jax.experimental.pallas import tpu_sc as plsc

_V = 1_000_000
_D = 32
_FLAT = 16384 * 50

_ROWBLK = 8000


def _transform_body(t_ref, wt_ref, b_ref, o_ref):
    o_ref[...] = (
        jnp.dot(t_ref[...], wt_ref[...], preferred_element_type=jnp.float32)
        + b_ref[...]
    )


def _transform(table, Wt, b2d):
    n = table.shape[0]
    return pl.pallas_call(
        _transform_body,
        grid=(n // _ROWBLK,),
        in_specs=[
            pl.BlockSpec((_ROWBLK, _D), lambda i: (i, 0)),
            pl.BlockSpec((_D, _D), lambda i: (0, 0)),
            pl.BlockSpec((1, _D), lambda i: (0, 0)),
        ],
        out_specs=pl.BlockSpec((_ROWBLK, _D), lambda i: (i, 0)),
        out_shape=jax.ShapeDtypeStruct((n, _D), jnp.float32),
    )(table, Wt, b2d)


_NW = 32
_PERW = _FLAT // _NW
_CH = 640
_NCH = _PERW // _CH
_NBUF = 4
_NGRP = _NCH // _NBUF


@functools.cache
def _build_gather():
    mesh = plsc.VectorSubcoreMesh(core_axis_name="c", subcore_axis_name="s")

    @functools.partial(
        pl.kernel,
        mesh=mesh,
        compiler_params=pltpu.CompilerParams(use_tc_tiling_on_sc=False),
        out_type=jax.ShapeDtypeStruct((_FLAT, _D), jnp.float32),
        scratch_types=[
            pltpu.VMEM((_PERW,), jnp.int32),
            [pltpu.VMEM((_CH, _D), jnp.float32) for _ in range(_NBUF)],
            [pltpu.SemaphoreType.DMA for _ in range(_NBUF)],
        ],
    )
    def _gather(tt_hbm, idx_hbm, out_hbm, idx_v, rows, sems):
        wid = lax.axis_index("s") * 2 + lax.axis_index("c")
        base = wid * _PERW
        pltpu.sync_copy(idx_hbm.at[pl.ds(base, _PERW)], idx_v)
        for j in range(_NBUF):
            pltpu.async_copy(
                tt_hbm.at[idx_v.at[pl.ds(j * _CH, _CH)]], rows[j], sems[j]
            )

        def outer(g, carry):
            for j in range(_NBUF):
                c = g * _NBUF + j
                pltpu.make_async_copy(
                    tt_hbm.at[idx_v.at[pl.ds(0, _CH)]], rows[j], sems[j]
                ).wait()
                pltpu.sync_copy(
                    rows[j], out_hbm.at[pl.ds(base + c * _CH, _CH)]
                )
                nc = c + _NBUF

                @pl.when(nc < _NCH)
                def _():
                    pltpu.async_copy(
                        tt_hbm.at[idx_v.at[pl.ds(nc * _CH, _CH)]],
                        rows[j],
                        sems[j],
                    )
            return carry

        lax.fori_loop(0, _NGRP, outer, 0)

    return _gather


def kernel(idx, table, W, b):
    ttable = _transform(table, W.T, b.reshape(1, _D))
    flat = _build_gather()(ttable, idx.reshape(-1))
    return flat.reshape(idx.shape[0], idx.shape[1], _D)


# speedup vs baseline: 11.6733x; 1.1822x over previous
import functools

import jax
import jax.numpy as jnp
from jax import lax
from jax.experimental import pallas as pl
from jax.experimental.pallas import tpu as pltpu
from jax.experimental.pallas import tpu_sc as plsc

_V = 1_000_000
_D = 32
_FLAT = 16384 * 50

_ROWBLK = 2000


def _transform_body(t_ref, wbd_ref, b_ref, o_ref):
    o_ref[...] = (
        jnp.dot(t_ref[...], wbd_ref[...], preferred_element_type=jnp.float32)
        + b_ref[...]
    )


def _transform(tp, Wbd, b128):
    n = tp.shape[0]
    return pl.pallas_call(
        _transform_body,
        grid=(n // _ROWBLK,),
        in_specs=[
            pl.BlockSpec((_ROWBLK, 4 * _D), lambda i: (i, 0)),
            pl.BlockSpec((4 * _D, 4 * _D), lambda i: (0, 0)),
            pl.BlockSpec((1, 4 * _D), lambda i: (0, 0)),
        ],
        out_specs=pl.BlockSpec((_ROWBLK, 4 * _D), lambda i: (i, 0)),
        out_shape=jax.ShapeDtypeStruct((n, 4 * _D), jnp.float32),
    )(tp, Wbd, b128)


_NW = 32
_PERW = _FLAT // _NW
_CH = 640
_NCH = _PERW // _CH
_NBUF = 4
_NGRP = _NCH // _NBUF


@functools.cache
def _build_gather():
    mesh = plsc.VectorSubcoreMesh(core_axis_name="c", subcore_axis_name="s")

    @functools.partial(
        pl.kernel,
        mesh=mesh,
        compiler_params=pltpu.CompilerParams(use_tc_tiling_on_sc=False),
        out_type=jax.ShapeDtypeStruct((_FLAT, _D), jnp.float32),
        scratch_types=[
            pltpu.VMEM((_PERW,), jnp.int32),
            [pltpu.VMEM((_CH, _D), jnp.float32) for _ in range(_NBUF)],
            [pltpu.SemaphoreType.DMA for _ in range(_NBUF)],
        ],
    )
    def _gather(tt_hbm, idx_hbm, out_hbm, idx_v, rows, sems):
        wid = lax.axis_index("s") * 2 + lax.axis_index("c")
        base = wid * _PERW
        pltpu.sync_copy(idx_hbm.at[pl.ds(base, _PERW)], idx_v)
        for j in range(_NBUF):
            pltpu.async_copy(
                tt_hbm.at[idx_v.at[pl.ds(j * _CH, _CH)]], rows[j], sems[j]
            )

        def outer(g, carry):
            for j in range(_NBUF):
                c = g * _NBUF + j
                pltpu.make_async_copy(
                    tt_hbm.at[idx_v.at[pl.ds(0, _CH)]], rows[j], sems[j]
                ).wait()
                pltpu.sync_copy(
                    rows[j], out_hbm.at[pl.ds(base + c * _CH, _CH)]
                )
                nc = c + _NBUF

                @pl.when(nc < _NCH)
                def _():
                    pltpu.async_copy(
                        tt_hbm.at[idx_v.at[pl.ds(nc * _CH, _CH)]],
                        rows[j],
                        sems[j],
                    )
            return carry

        lax.fori_loop(0, _NGRP, outer, 0)

    return _gather


def kernel(idx, table, W, b):
    tp = jnp.reshape(table, (_V // 4, 4 * _D))
    eye4 = jnp.eye(4, dtype=jnp.float32)
    Wbd = jnp.einsum("pq,do->pdqo", eye4, W.T).reshape(4 * _D, 4 * _D)
    b128 = jnp.tile(b, 4).reshape(1, 4 * _D)
    ttp = _transform(tp, Wbd, b128)
    ttable = jnp.reshape(ttp, (_V, _D))
    flat = _build_gather()(ttable, idx.reshape(-1))
    return flat.reshape(idx.shape[0], idx.shape[1], _D)


# speedup vs baseline: 13.8478x; 1.1863x over previous
import functools

import jax
import jax.numpy as jnp
from jax import lax
from jax.experimental import pallas as pl
from jax.experimental.pallas import tpu as pltpu
from jax.experimental.pallas import tpu_sc as plsc

_V = 1_000_000
_D = 32
_B = 16384
_L = 50
_FLAT = _B * _L

_NW = 32
_PERW = _FLAT // _NW
_CH = 640
_NCH = _PERW // _CH
_NBUF = 4
_NGRP = _NCH // _NBUF


@functools.cache
def _build_gather():
    mesh = plsc.VectorSubcoreMesh(core_axis_name="c", subcore_axis_name="s")

    @functools.partial(
        pl.kernel,
        mesh=mesh,
        compiler_params=pltpu.CompilerParams(use_tc_tiling_on_sc=False),
        out_type=jax.ShapeDtypeStruct((_FLAT, _D), jnp.float32),
        scratch_types=[
            pltpu.VMEM((_PERW,), jnp.int32),
            [pltpu.VMEM((_CH, _D), jnp.float32) for _ in range(_NBUF)],
            [pltpu.SemaphoreType.DMA for _ in range(_NBUF)],
        ],
    )
    def _gather(tt_hbm, idx_hbm, out_hbm, idx_v, rows, sems):
        wid = lax.axis_index("s") * 2 + lax.axis_index("c")
        base = wid * _PERW
        pltpu.sync_copy(idx_hbm.at[pl.ds(base, _PERW)], idx_v)
        for j in range(_NBUF):
            pltpu.async_copy(
                tt_hbm.at[idx_v.at[pl.ds(j * _CH, _CH)]], rows[j], sems[j]
            )

        def outer(g, carry):
            for j in range(_NBUF):
                c = g * _NBUF + j
                pltpu.make_async_copy(
                    tt_hbm.at[idx_v.at[pl.ds(0, _CH)]], rows[j], sems[j]
                ).wait()
                pltpu.sync_copy(
                    rows[j], out_hbm.at[pl.ds(base + c * _CH, _CH)]
                )
                nc = c + _NBUF

                @pl.when(nc < _NCH)
                def _():
                    pltpu.async_copy(
                        tt_hbm.at[idx_v.at[pl.ds(nc * _CH, _CH)]],
                        rows[j],
                        sems[j],
                    )
            return carry

        lax.fori_loop(0, _NGRP, outer, 0)

    return _gather


_BB = 64


def _decode_body(r_ref, wt_ref, b_ref, o_ref):
    y = (
        jnp.dot(r_ref[...], wt_ref[...], preferred_element_type=jnp.float32)
        + b_ref[...]
    )
    o_ref[...] = y.reshape(_BB, _L, _D)


def _decode(raw, Wt, b2d):
    return pl.pallas_call(
        _decode_body,
        grid=(_B // _BB,),
        in_specs=[
            pl.BlockSpec((_BB * _L, _D), lambda i: (i, 0)),
            pl.BlockSpec((_D, _D), lambda i: (0, 0)),
            pl.BlockSpec((1, _D), lambda i: (0, 0)),
        ],
        out_specs=pl.BlockSpec((_BB, _L, _D), lambda i: (i, 0, 0)),
        out_shape=jax.ShapeDtypeStruct((_B, _L, _D), jnp.float32),
    )(raw, Wt, b2d)


def kernel(idx, table, W, b):
    raw = _build_gather()(table, idx.reshape(-1))
    return _decode(raw, W.T, b.reshape(1, _D))
